# Initial kernel scaffold; baseline (speedup 1.0000x reference)
import jax
import jax.numpy as jnp
from jax.experimental import pallas as pl


def kernel(x, edge_index, edge_attr, batch, msg_W1, msg_b1, msg_W2, msg_b2, msg_W3, msg_b3, node_W1, node_b1, node_W2, node_b2, node_W3, node_b3, glob_W1, glob_b1, glob_W2, glob_b2, glob_W3, glob_b3):
    raise NotImplementedError("write your pallas kernel here")


# speedup vs baseline: 1.7267x; 1.7267x over previous
import functools

import jax
import jax.numpy as jnp
from jax import lax
from jax.experimental import pallas as pl
from jax.experimental.pallas import tpu as pltpu
from jax.experimental.pallas import tpu_sc as plsc

_N = 10000
_E = 320000
_NF = 128
_EF = 16
_MSG = 128
_HID = 300
_NH = 128
_NP = 1
_G = 64

_NC = 2
_NS = 16
_NW = _NC * _NS

_S = 128
_ROWS_PER_TILE = 80
_EPAD = _NW * _ROWS_PER_TILE * _S
_IDX_BLK = 16
_NPAD = 10240
_DEAD_ROW = 10200
_RPT = _NPAD // _NS

_BE = 2560
_BN = 1000


def _bdot(a, b):
  return jax.lax.dot(a.astype(jnp.bfloat16), b.astype(jnp.bfloat16),
                     preferred_element_type=jnp.float32)


def _sc_gather(x, src3, dst3):
  mesh = plsc.VectorSubcoreMesh(core_axis_name="c", subcore_axis_name="s")

  @functools.partial(
      pl.kernel, mesh=mesh,
      out_type=(jax.ShapeDtypeStruct((_EPAD, _NF), jnp.float32),
                jax.ShapeDtypeStruct((_EPAD, _NF), jnp.float32)),
      scratch_types=[
          pltpu.VMEM((_IDX_BLK, _S), jnp.int32),
          pltpu.VMEM((_IDX_BLK, _S), jnp.int32),
          pltpu.VMEM((_S, _NF), jnp.float32),
          pltpu.VMEM((_S, _NF), jnp.float32),
          pltpu.SemaphoreType.DMA,
          pltpu.SemaphoreType.DMA,
      ],
  )
  def k(x_hbm, src_hbm, dst_hbm, xi_hbm, xj_hbm,
        si_v, di_v, ri_v, rj_v, sem_i, sem_j):
    wid = lax.axis_index("s") * _NC + lax.axis_index("c")
    row0 = wid * _ROWS_PER_TILE

    @pl.loop(0, _ROWS_PER_TILE, step=_IDX_BLK)
    def _(j2):
      pltpu.sync_copy(dst_hbm.at[pl.ds(row0 + j2, _IDX_BLK)], di_v)
      pltpu.sync_copy(src_hbm.at[pl.ds(row0 + j2, _IDX_BLK)], si_v)

      @pl.loop(0, _IDX_BLK)
      def _(j):
        cp_i = pltpu.async_copy(x_hbm.at[di_v.at[j]], ri_v, sem_i)
        cp_j = pltpu.async_copy(x_hbm.at[si_v.at[j]], rj_v, sem_j)
        cp_i.wait()
        cp_j.wait()
        eoff = (row0 + j2 + j) * _S
        pltpu.sync_copy(ri_v, xi_hbm.at[pl.ds(eoff, _S)])
        pltpu.sync_copy(rj_v, xj_hbm.at[pl.ds(eoff, _S)])

  return k(x, src3, dst3)


def _sc_scatter_add(msg, dst3, zrows):
  mesh = plsc.VectorSubcoreMesh(core_axis_name="c", subcore_axis_name="s")

  @functools.partial(
      pl.kernel, mesh=mesh,
      out_type=(jax.ShapeDtypeStruct((_NPAD, _MSG), jnp.float32),
                jax.ShapeDtypeStruct((_NPAD, _MSG), jnp.float32)),
      scratch_types=[
          pltpu.VMEM((_IDX_BLK, _S), jnp.int32),
          pltpu.VMEM((_S, _MSG), jnp.float32),
          pltpu.VMEM_SHARED((_NPAD, _MSG), jnp.float32),
      ],
  )
  def k(msg_hbm, dst_hbm, z_hbm, out0_hbm, out1_hbm,
        di_v, rows_v, acc_sh):
    cid = lax.axis_index("c")
    sid = lax.axis_index("s")
    wid = sid * _NC + cid
    row0 = wid * _ROWS_PER_TILE

    pltpu.sync_copy(z_hbm, acc_sh.at[pl.ds(sid * _RPT, _RPT)])
    plsc.subcore_barrier()

    @pl.loop(0, _ROWS_PER_TILE, step=_IDX_BLK)
    def _(j2):
      pltpu.sync_copy(dst_hbm.at[pl.ds(row0 + j2, _IDX_BLK)], di_v)

      @pl.loop(0, _IDX_BLK)
      def _(j):
        eoff = (row0 + j2 + j) * _S
        pltpu.sync_copy(msg_hbm.at[pl.ds(eoff, _S)], rows_v)
        pltpu.sync_copy(rows_v, acc_sh.at[di_v.at[j]], add=True)

    plsc.subcore_barrier()

    @pl.when(cid == 0)
    def _():
      pltpu.sync_copy(acc_sh.at[pl.ds(sid * _RPT, _RPT)],
                      out0_hbm.at[pl.ds(sid * _RPT, _RPT)])

    @pl.when(cid == 1)
    def _():
      pltpu.sync_copy(acc_sh.at[pl.ds(sid * _RPT, _RPT)],
                      out1_hbm.at[pl.ds(sid * _RPT, _RPT)])

  return k(msg, dst3, zrows)


def _edge_mlp(xi, xj, ea, W1a, W1b, W1c, b1, W2, b2, W3, b3):
  grid = (_EPAD // _BE,)

  def body(xi_ref, xj_ref, ea_ref, W1a_ref, W1b_ref, W1c_ref, b1_ref,
           W2_ref, b2_ref, W3_ref, b3_ref, out_ref):
    cat = jnp.concatenate([xi_ref[...], xj_ref[...], ea_ref[...]], axis=1)
    W1 = jnp.concatenate([W1a_ref[...], W1b_ref[...], W1c_ref[...]], axis=0)
    h = _bdot(cat, W1) + b1_ref[...]
    h = jnp.maximum(h, 0.0)
    h = jnp.maximum(_bdot(h, W2_ref[...]) + b2_ref[...], 0.0)
    out_ref[...] = _bdot(h, W3_ref[...]) + b3_ref[...]

  full = lambda shape: pl.BlockSpec(shape, lambda i: (0, 0))
  return pl.pallas_call(
      body,
      grid=grid,
      in_specs=[
          pl.BlockSpec((_BE, _NF), lambda i: (i, 0)),
          pl.BlockSpec((_BE, _NF), lambda i: (i, 0)),
          pl.BlockSpec((_BE, _EF), lambda i: (i, 0)),
          full((_NF, _HID)),
          full((_NF, _HID)),
          full((_EF, _HID)),
          full((1, _HID)),
          full((_HID, _HID)),
          full((1, _HID)),
          full((_HID, _MSG)),
          full((1, _MSG)),
      ],
      out_specs=pl.BlockSpec((_BE, _MSG), lambda i: (i, 0)),
      out_shape=jax.ShapeDtypeStruct((_EPAD, _MSG), jnp.float32),
  )(xi, xj, ea, W1a, W1b, W1c, b1, W2, b2, W3, b3)


def _node_pool(x, a0, a1, batch3,
               Wn1a, Wn1b, bn1, Wn2, bn2, Wn3, bn3,
               Wg1, bg1, Wg2, bg2, Wg3, bg3):
  grid = (_N // _BN,)
  nsteps = _N // _BN

  def body(x_ref, a0_ref, a1_ref, b_ref,
           Wn1a_ref, Wn1b_ref, bn1_ref, Wn2_ref, bn2_ref, Wn3_ref, bn3_ref,
           Wg1_ref, bg1_ref, Wg2_ref, bg2_ref, Wg3_ref, bg3_ref,
           out_ref, sums_ref, cnt_ref):
    i = pl.program_id(0)

    @pl.when(i == 0)
    def _():
      sums_ref[...] = jnp.zeros_like(sums_ref)
      cnt_ref[...] = jnp.zeros_like(cnt_ref)

    aggr = a0_ref[...] + a1_ref[...]
    cat = jnp.concatenate([x_ref[...], aggr], axis=1)
    Wn1 = jnp.concatenate([Wn1a_ref[...], Wn1b_ref[...]], axis=0)
    h = _bdot(cat, Wn1) + bn1_ref[...]
    h = jnp.maximum(h, 0.0)
    h = jnp.maximum(_bdot(h, Wn2_ref[...]) + bn2_ref[...], 0.0)
    h = _bdot(h, Wn3_ref[...]) + bn3_ref[...]

    bids = b_ref[0, 0, :]
    gids = lax.broadcasted_iota(jnp.int32, (1, _G), 1)
    oh = (bids[:, None] == gids).astype(jnp.float32)
    dn = (((0,), (0,)), ((), ()))
    h1 = h.astype(jnp.bfloat16).astype(jnp.float32)
    r1 = h - h1
    h2 = r1.astype(jnp.bfloat16).astype(jnp.float32)
    h3 = r1 - h2
    sums_ref[...] += (lax.dot_general(oh, h1, dn)
                      + lax.dot_general(oh, h2, dn)
                      + lax.dot_general(oh, h3, dn))
    ones = jnp.ones((_BN, _MSG), jnp.float32)
    cnt_ref[...] += lax.dot_general(oh, ones, dn)

    @pl.when(i == nsteps - 1)
    def _():
      pooled = sums_ref[...] / jnp.maximum(cnt_ref[...], 1.0)
      g = jnp.maximum(_bdot(pooled, Wg1_ref[...]) + bg1_ref[...], 0.0)
      g = jnp.maximum(_bdot(g, Wg2_ref[...]) + bg2_ref[...], 0.0)
      out_ref[...] = _bdot(g, Wg3_ref[...]) + bg3_ref[...]

  full = lambda shape: pl.BlockSpec(shape, lambda i: tuple(0 for _ in shape))
  return pl.pallas_call(
      body,
      grid=grid,
      in_specs=[
          pl.BlockSpec((_BN, _NF), lambda i: (i, 0)),
          pl.BlockSpec((_BN, _MSG), lambda i: (i, 0)),
          pl.BlockSpec((_BN, _MSG), lambda i: (i, 0)),
          pl.BlockSpec((1, 1, _BN), lambda i: (i, 0, 0)),
          full((_NF, _HID)),
          full((_MSG, _HID)),
          full((1, _HID)),
          full((_HID, _HID)),
          full((1, _HID)),
          full((_HID, _NH)),
          full((1, _NH)),
          full((_NH, _NH)),
          full((1, _NH)),
          full((_NH, _NH)),
          full((1, _NH)),
          full((_NH, _NP)),
          full((1, _NP)),
      ],
      out_specs=pl.BlockSpec((_G, _NP), lambda i: (0, 0)),
      out_shape=jax.ShapeDtypeStruct((_G, _NP), jnp.float32),
      scratch_shapes=[
          pltpu.VMEM((_G, _MSG), jnp.float32),
          pltpu.VMEM((_G, _MSG), jnp.float32),
      ],
  )(x, a0, a1, batch3,
    Wn1a, Wn1b, bn1, Wn2, bn2, Wn3, bn3,
    Wg1, bg1, Wg2, bg2, Wg3, bg3)


def kernel(x, edge_index, edge_attr, batch,
           msg_W1, msg_b1, msg_W2, msg_b2, msg_W3, msg_b3,
           node_W1, node_b1, node_W2, node_b2, node_W3, node_b3,
           glob_W1, glob_b1, glob_W2, glob_b2, glob_W3, glob_b3):
  src = edge_index[0]
  dst = edge_index[1]
  pad = _EPAD - _E
  src3 = jnp.concatenate(
      [src, jnp.zeros((pad,), jnp.int32)]).reshape(_EPAD // _S, _S)
  dst3g = jnp.concatenate(
      [dst, jnp.zeros((pad,), jnp.int32)]).reshape(_EPAD // _S, _S)
  dst3s = jnp.concatenate(
      [dst, jnp.full((pad,), _DEAD_ROW, jnp.int32)]).reshape(_EPAD // _S, _S)
  ea = jnp.concatenate([edge_attr, jnp.zeros((pad, _EF), jnp.float32)], axis=0)

  xi, xj = _sc_gather(x, src3, dst3g)

  msg = _edge_mlp(
      xi, xj, ea,
      msg_W1[:_NF], msg_W1[_NF:2 * _NF], msg_W1[2 * _NF:],
      msg_b1.reshape(1, _HID),
      msg_W2, msg_b2.reshape(1, _HID),
      msg_W3, msg_b3.reshape(1, _MSG))

  zrows = jnp.zeros((_RPT, _MSG), jnp.float32)
  a0, a1 = _sc_scatter_add(msg, dst3s, zrows)

  out = _node_pool(
      x, a0, a1, batch.reshape(_N // _BN, 1, _BN),
      node_W1[:_NF], node_W1[_NF:], node_b1.reshape(1, _HID),
      node_W2, node_b2.reshape(1, _HID),
      node_W3, node_b3.reshape(1, _NH),
      glob_W1, glob_b1.reshape(1, _NH),
      glob_W2, glob_b2.reshape(1, _NH),
      glob_W3, glob_b3.reshape(1, _NP))
  return out


# speedup vs baseline: 1.8379x; 1.0644x over previous
import functools

import jax
import jax.numpy as jnp
from jax import lax
from jax.experimental import pallas as pl
from jax.experimental.pallas import tpu as pltpu
from jax.experimental.pallas import tpu_sc as plsc

_N = 10000
_E = 320000
_NF = 128
_EF = 16
_MSG = 128
_HID = 300
_NH = 128
_NP = 1
_G = 64

_NC = 2
_NS = 16
_NW = _NC * _NS

_S = 128
_ROWS_PER_TILE = 80
_EPAD = _NW * _ROWS_PER_TILE * _S
_IDX_BLK = 16
_NPAD = 10240
_DEAD_ROW = 10200
_RPT = _NPAD // _NS

_BE = 2560
_BN = 1000


def _bdot(a, b):
  return jax.lax.dot(a.astype(jnp.bfloat16), b.astype(jnp.bfloat16),
                     preferred_element_type=jnp.float32)


def _sc_gather(x, src3, dst3):
  mesh = plsc.VectorSubcoreMesh(core_axis_name="c", subcore_axis_name="s")

  @functools.partial(
      pl.kernel, mesh=mesh,
      out_type=(jax.ShapeDtypeStruct((_EPAD, _NF), jnp.float32),
                jax.ShapeDtypeStruct((_EPAD, _NF), jnp.float32)),
      scratch_types=[
          pltpu.VMEM((_ROWS_PER_TILE, _S), jnp.int32),
          pltpu.VMEM((_ROWS_PER_TILE, _S), jnp.int32),
          pltpu.VMEM((_S, _NF), jnp.float32),
          pltpu.VMEM((_S, _NF), jnp.float32),
          pltpu.VMEM((_S, _NF), jnp.float32),
          pltpu.VMEM((_S, _NF), jnp.float32),
          pltpu.SemaphoreType.DMA, pltpu.SemaphoreType.DMA,
          pltpu.SemaphoreType.DMA, pltpu.SemaphoreType.DMA,
          pltpu.SemaphoreType.DMA, pltpu.SemaphoreType.DMA,
          pltpu.SemaphoreType.DMA, pltpu.SemaphoreType.DMA,
      ],
  )
  def k(x_hbm, src_hbm, dst_hbm, xi_hbm, xj_hbm,
        si_v, di_v, ri0, ri1, rj0, rj1,
        sgi0, sgi1, sgj0, sgj1, swi0, swi1, swj0, swj1):
    wid = lax.axis_index("s") * _NC + lax.axis_index("c")
    row0 = wid * _ROWS_PER_TILE
    NR = _ROWS_PER_TILE
    ri = (ri0, ri1)
    rj = (rj0, rj1)
    sgi = (sgi0, sgi1)
    sgj = (sgj0, sgj1)
    swi = (swi0, swi1)
    swj = (swj0, swj1)

    pltpu.sync_copy(dst_hbm.at[pl.ds(row0, NR)], di_v)
    pltpu.sync_copy(src_hbm.at[pl.ds(row0, NR)], si_v)

    pltpu.async_copy(x_hbm.at[di_v.at[0]], ri0, sgi0)
    pltpu.async_copy(x_hbm.at[si_v.at[0]], rj0, sgj0)

    @pl.loop(0, NR, step=2)
    def _(j):
      for b in range(2):
        jj = j + b
        nb = 1 - b

        @pl.when(jj + 1 < NR)
        def _():
          @pl.when(jj >= 1)
          def _():
            eo = (row0 + jj - 1) * _S
            pltpu.make_async_copy(
                ri[nb], xi_hbm.at[pl.ds(eo, _S)], swi[nb]).wait()
            pltpu.make_async_copy(
                rj[nb], xj_hbm.at[pl.ds(eo, _S)], swj[nb]).wait()
          pltpu.async_copy(x_hbm.at[di_v.at[jj + 1]], ri[nb], sgi[nb])
          pltpu.async_copy(x_hbm.at[si_v.at[jj + 1]], rj[nb], sgj[nb])

        pltpu.make_async_copy(x_hbm.at[di_v.at[jj]], ri[b], sgi[b]).wait()
        pltpu.make_async_copy(x_hbm.at[si_v.at[jj]], rj[b], sgj[b]).wait()
        eoff = (row0 + jj) * _S
        pltpu.async_copy(ri[b], xi_hbm.at[pl.ds(eoff, _S)], swi[b])
        pltpu.async_copy(rj[b], xj_hbm.at[pl.ds(eoff, _S)], swj[b])

    eo0 = (row0 + NR - 2) * _S
    eo1 = (row0 + NR - 1) * _S
    pltpu.make_async_copy(ri0, xi_hbm.at[pl.ds(eo0, _S)], swi0).wait()
    pltpu.make_async_copy(rj0, xj_hbm.at[pl.ds(eo0, _S)], swj0).wait()
    pltpu.make_async_copy(ri1, xi_hbm.at[pl.ds(eo1, _S)], swi1).wait()
    pltpu.make_async_copy(rj1, xj_hbm.at[pl.ds(eo1, _S)], swj1).wait()

  return k(x, src3, dst3)


def _sc_scatter_add(msg, dst3, zrows):
  mesh = plsc.VectorSubcoreMesh(core_axis_name="c", subcore_axis_name="s")

  @functools.partial(
      pl.kernel, mesh=mesh,
      out_type=(jax.ShapeDtypeStruct((_NPAD, _MSG), jnp.float32),
                jax.ShapeDtypeStruct((_NPAD, _MSG), jnp.float32)),
      scratch_types=[
          pltpu.VMEM((_IDX_BLK, _S), jnp.int32),
          pltpu.VMEM((_S, _MSG), jnp.float32),
          pltpu.VMEM_SHARED((_NPAD, _MSG), jnp.float32),
      ],
  )
  def k(msg_hbm, dst_hbm, z_hbm, out0_hbm, out1_hbm,
        di_v, rows_v, acc_sh):
    cid = lax.axis_index("c")
    sid = lax.axis_index("s")
    wid = sid * _NC + cid
    row0 = wid * _ROWS_PER_TILE

    pltpu.sync_copy(z_hbm, acc_sh.at[pl.ds(sid * _RPT, _RPT)])
    plsc.subcore_barrier()

    @pl.loop(0, _ROWS_PER_TILE, step=_IDX_BLK)
    def _(j2):
      pltpu.sync_copy(dst_hbm.at[pl.ds(row0 + j2, _IDX_BLK)], di_v)

      @pl.loop(0, _IDX_BLK)
      def _(j):
        eoff = (row0 + j2 + j) * _S
        pltpu.sync_copy(msg_hbm.at[pl.ds(eoff, _S)], rows_v)
        pltpu.sync_copy(rows_v, acc_sh.at[di_v.at[j]], add=True)

    plsc.subcore_barrier()

    @pl.when(cid == 0)
    def _():
      pltpu.sync_copy(acc_sh.at[pl.ds(sid * _RPT, _RPT)],
                      out0_hbm.at[pl.ds(sid * _RPT, _RPT)])

    @pl.when(cid == 1)
    def _():
      pltpu.sync_copy(acc_sh.at[pl.ds(sid * _RPT, _RPT)],
                      out1_hbm.at[pl.ds(sid * _RPT, _RPT)])

  return k(msg, dst3, zrows)


def _edge_mlp(xi, xj, ea, W1a, W1b, W1c, b1, W2, b2, W3, b3):
  grid = (_EPAD // _BE,)

  def body(xi_ref, xj_ref, ea_ref, W1a_ref, W1b_ref, W1c_ref, b1_ref,
           W2_ref, b2_ref, W3_ref, b3_ref, out_ref):
    cat = jnp.concatenate([xi_ref[...], xj_ref[...], ea_ref[...]], axis=1)
    W1 = jnp.concatenate([W1a_ref[...], W1b_ref[...], W1c_ref[...]], axis=0)
    h = _bdot(cat, W1) + b1_ref[...]
    h = jnp.maximum(h, 0.0)
    h = jnp.maximum(_bdot(h, W2_ref[...]) + b2_ref[...], 0.0)
    out_ref[...] = _bdot(h, W3_ref[...]) + b3_ref[...]

  full = lambda shape: pl.BlockSpec(shape, lambda i: (0, 0))
  return pl.pallas_call(
      body,
      grid=grid,
      in_specs=[
          pl.BlockSpec((_BE, _NF), lambda i: (i, 0)),
          pl.BlockSpec((_BE, _NF), lambda i: (i, 0)),
          pl.BlockSpec((_BE, _EF), lambda i: (i, 0)),
          full((_NF, _HID)),
          full((_NF, _HID)),
          full((_EF, _HID)),
          full((1, _HID)),
          full((_HID, _HID)),
          full((1, _HID)),
          full((_HID, _MSG)),
          full((1, _MSG)),
      ],
      out_specs=pl.BlockSpec((_BE, _MSG), lambda i: (i, 0)),
      out_shape=jax.ShapeDtypeStruct((_EPAD, _MSG), jnp.float32),
  )(xi, xj, ea, W1a, W1b, W1c, b1, W2, b2, W3, b3)


def _node_pool(x, a0, a1, batch3,
               Wn1a, Wn1b, bn1, Wn2, bn2, Wn3, bn3,
               Wg1, bg1, Wg2, bg2, Wg3, bg3):
  grid = (_N // _BN,)
  nsteps = _N // _BN

  def body(x_ref, a0_ref, a1_ref, b_ref,
           Wn1a_ref, Wn1b_ref, bn1_ref, Wn2_ref, bn2_ref, Wn3_ref, bn3_ref,
           Wg1_ref, bg1_ref, Wg2_ref, bg2_ref, Wg3_ref, bg3_ref,
           out_ref, sums_ref, cnt_ref):
    i = pl.program_id(0)

    @pl.when(i == 0)
    def _():
      sums_ref[...] = jnp.zeros_like(sums_ref)
      cnt_ref[...] = jnp.zeros_like(cnt_ref)

    aggr = a0_ref[...] + a1_ref[...]
    cat = jnp.concatenate([x_ref[...], aggr], axis=1)
    Wn1 = jnp.concatenate([Wn1a_ref[...], Wn1b_ref[...]], axis=0)
    h = _bdot(cat, Wn1) + bn1_ref[...]
    h = jnp.maximum(h, 0.0)
    h = jnp.maximum(_bdot(h, Wn2_ref[...]) + bn2_ref[...], 0.0)
    h = _bdot(h, Wn3_ref[...]) + bn3_ref[...]

    bids = b_ref[0, 0, :]
    gids = lax.broadcasted_iota(jnp.int32, (1, _G), 1)
    oh = (bids[:, None] == gids).astype(jnp.float32)
    dn = (((0,), (0,)), ((), ()))
    h1 = h.astype(jnp.bfloat16).astype(jnp.float32)
    r1 = h - h1
    h2 = r1.astype(jnp.bfloat16).astype(jnp.float32)
    h3 = r1 - h2
    sums_ref[...] += (lax.dot_general(oh, h1, dn)
                      + lax.dot_general(oh, h2, dn)
                      + lax.dot_general(oh, h3, dn))
    ones = jnp.ones((_BN, _MSG), jnp.float32)
    cnt_ref[...] += lax.dot_general(oh, ones, dn)

    @pl.when(i == nsteps - 1)
    def _():
      pooled = sums_ref[...] / jnp.maximum(cnt_ref[...], 1.0)
      g = jnp.maximum(_bdot(pooled, Wg1_ref[...]) + bg1_ref[...], 0.0)
      g = jnp.maximum(_bdot(g, Wg2_ref[...]) + bg2_ref[...], 0.0)
      out_ref[...] = _bdot(g, Wg3_ref[...]) + bg3_ref[...]

  full = lambda shape: pl.BlockSpec(shape, lambda i: tuple(0 for _ in shape))
  return pl.pallas_call(
      body,
      grid=grid,
      in_specs=[
          pl.BlockSpec((_BN, _NF), lambda i: (i, 0)),
          pl.BlockSpec((_BN, _MSG), lambda i: (i, 0)),
          pl.BlockSpec((_BN, _MSG), lambda i: (i, 0)),
          pl.BlockSpec((1, 1, _BN), lambda i: (i, 0, 0)),
          full((_NF, _HID)),
          full((_MSG, _HID)),
          full((1, _HID)),
          full((_HID, _HID)),
          full((1, _HID)),
          full((_HID, _NH)),
          full((1, _NH)),
          full((_NH, _NH)),
          full((1, _NH)),
          full((_NH, _NH)),
          full((1, _NH)),
          full((_NH, _NP)),
          full((1, _NP)),
      ],
      out_specs=pl.BlockSpec((_G, _NP), lambda i: (0, 0)),
      out_shape=jax.ShapeDtypeStruct((_G, _NP), jnp.float32),
      scratch_shapes=[
          pltpu.VMEM((_G, _MSG), jnp.float32),
          pltpu.VMEM((_G, _MSG), jnp.float32),
      ],
  )(x, a0, a1, batch3,
    Wn1a, Wn1b, bn1, Wn2, bn2, Wn3, bn3,
    Wg1, bg1, Wg2, bg2, Wg3, bg3)


def kernel(x, edge_index, edge_attr, batch,
           msg_W1, msg_b1, msg_W2, msg_b2, msg_W3, msg_b3,
           node_W1, node_b1, node_W2, node_b2, node_W3, node_b3,
           glob_W1, glob_b1, glob_W2, glob_b2, glob_W3, glob_b3):
  src = edge_index[0]
  dst = edge_index[1]
  pad = _EPAD - _E
  src3 = jnp.concatenate(
      [src, jnp.zeros((pad,), jnp.int32)]).reshape(_EPAD // _S, _S)
  dst3g = jnp.concatenate(
      [dst, jnp.zeros((pad,), jnp.int32)]).reshape(_EPAD // _S, _S)
  dst3s = jnp.concatenate(
      [dst, jnp.full((pad,), _DEAD_ROW, jnp.int32)]).reshape(_EPAD // _S, _S)
  ea = jnp.concatenate([edge_attr, jnp.zeros((pad, _EF), jnp.float32)], axis=0)

  xi, xj = _sc_gather(x, src3, dst3g)

  msg = _edge_mlp(
      xi, xj, ea,
      msg_W1[:_NF], msg_W1[_NF:2 * _NF], msg_W1[2 * _NF:],
      msg_b1.reshape(1, _HID),
      msg_W2, msg_b2.reshape(1, _HID),
      msg_W3, msg_b3.reshape(1, _MSG))

  zrows = jnp.zeros((_RPT, _MSG), jnp.float32)
  a0, a1 = _sc_scatter_add(msg, dst3s, zrows)

  out = _node_pool(
      x, a0, a1, batch.reshape(_N // _BN, 1, _BN),
      node_W1[:_NF], node_W1[_NF:], node_b1.reshape(1, _HID),
      node_W2, node_b2.reshape(1, _HID),
      node_W3, node_b3.reshape(1, _NH),
      glob_W1, glob_b1.reshape(1, _NH),
      glob_W2, glob_b2.reshape(1, _NH),
      glob_W3, glob_b3.reshape(1, _NP))
  return out


# speedup vs baseline: 2.1235x; 1.1554x over previous
import functools

import jax
import jax.numpy as jnp
from jax import lax
from jax.experimental import pallas as pl
from jax.experimental.pallas import tpu as pltpu
from jax.experimental.pallas import tpu_sc as plsc

_N = 10000
_E = 320000
_NF = 128
_EF = 16
_MSG = 128
_HID = 300
_NH = 128
_NP = 1
_G = 64

_NC = 2
_NS = 16
_NW = _NC * _NS

_S = 128
_ROWS_PER_TILE = 80
_EPAD = _NW * _ROWS_PER_TILE * _S
_IDX_BLK = 8
_NPAD = 10240
_DEAD_ROW = 10200
_RPT = _NPAD // _NS

_BE = 2560
_BN = 1000


def _bdot(a, b):
  return jax.lax.dot(a.astype(jnp.bfloat16), b.astype(jnp.bfloat16),
                     preferred_element_type=jnp.float32)


def _sc_gather(x, src3, dst3):
  mesh = plsc.VectorSubcoreMesh(core_axis_name="c", subcore_axis_name="s")
  nrows = src3.shape[0]
  epad = nrows * _S
  rpt = nrows // _NW

  @functools.partial(
      pl.kernel, mesh=mesh,
      out_type=(jax.ShapeDtypeStruct((epad, _NF), jnp.float32),
                jax.ShapeDtypeStruct((epad, _NF), jnp.float32)),
      scratch_types=[
          pltpu.VMEM((rpt, _S), jnp.int32),
          pltpu.VMEM((rpt, _S), jnp.int32),
          pltpu.VMEM((_S, _NF), jnp.float32),
          pltpu.VMEM((_S, _NF), jnp.float32),
          pltpu.VMEM((_S, _NF), jnp.float32),
          pltpu.VMEM((_S, _NF), jnp.float32),
          pltpu.SemaphoreType.DMA, pltpu.SemaphoreType.DMA,
          pltpu.SemaphoreType.DMA, pltpu.SemaphoreType.DMA,
          pltpu.SemaphoreType.DMA, pltpu.SemaphoreType.DMA,
          pltpu.SemaphoreType.DMA, pltpu.SemaphoreType.DMA,
      ],
  )
  def k(x_hbm, src_hbm, dst_hbm, xi_hbm, xj_hbm,
        si_v, di_v, ri0, ri1, rj0, rj1,
        sgi0, sgi1, sgj0, sgj1, swi0, swi1, swj0, swj1):
    wid = lax.axis_index("s") * _NC + lax.axis_index("c")
    row0 = wid * rpt
    NR = rpt
    ri = (ri0, ri1)
    rj = (rj0, rj1)
    sgi = (sgi0, sgi1)
    sgj = (sgj0, sgj1)
    swi = (swi0, swi1)
    swj = (swj0, swj1)

    pltpu.sync_copy(dst_hbm.at[pl.ds(row0, NR)], di_v)
    pltpu.sync_copy(src_hbm.at[pl.ds(row0, NR)], si_v)

    pltpu.async_copy(x_hbm.at[di_v.at[0]], ri0, sgi0)
    pltpu.async_copy(x_hbm.at[si_v.at[0]], rj0, sgj0)

    @pl.loop(0, NR, step=2)
    def _(j):
      for b in range(2):
        jj = j + b
        nb = 1 - b

        @pl.when(jj + 1 < NR)
        def _():
          @pl.when(jj >= 1)
          def _():
            eo = (row0 + jj - 1) * _S
            pltpu.make_async_copy(
                ri[nb], xi_hbm.at[pl.ds(eo, _S)], swi[nb]).wait()
            pltpu.make_async_copy(
                rj[nb], xj_hbm.at[pl.ds(eo, _S)], swj[nb]).wait()
          pltpu.async_copy(x_hbm.at[di_v.at[jj + 1]], ri[nb], sgi[nb])
          pltpu.async_copy(x_hbm.at[si_v.at[jj + 1]], rj[nb], sgj[nb])

        pltpu.make_async_copy(x_hbm.at[di_v.at[jj]], ri[b], sgi[b]).wait()
        pltpu.make_async_copy(x_hbm.at[si_v.at[jj]], rj[b], sgj[b]).wait()
        eoff = (row0 + jj) * _S
        pltpu.async_copy(ri[b], xi_hbm.at[pl.ds(eoff, _S)], swi[b])
        pltpu.async_copy(rj[b], xj_hbm.at[pl.ds(eoff, _S)], swj[b])

    eo0 = (row0 + NR - 2) * _S
    eo1 = (row0 + NR - 1) * _S
    pltpu.make_async_copy(ri0, xi_hbm.at[pl.ds(eo0, _S)], swi0).wait()
    pltpu.make_async_copy(rj0, xj_hbm.at[pl.ds(eo0, _S)], swj0).wait()
    pltpu.make_async_copy(ri1, xi_hbm.at[pl.ds(eo1, _S)], swi1).wait()
    pltpu.make_async_copy(rj1, xj_hbm.at[pl.ds(eo1, _S)], swj1).wait()

  return k(x, src3, dst3)


def _sc_scatter_add(msg, dst3, zrows):
  mesh = plsc.VectorSubcoreMesh(core_axis_name="c", subcore_axis_name="s")
  nrows = dst3.shape[0]
  rpt = nrows // _NW

  @functools.partial(
      pl.kernel, mesh=mesh,
      out_type=(jax.ShapeDtypeStruct((_NPAD, _MSG), jnp.float32),
                jax.ShapeDtypeStruct((_NPAD, _MSG), jnp.float32)),
      scratch_types=[
          pltpu.VMEM((_IDX_BLK, _S), jnp.int32),
          pltpu.VMEM((_S, _MSG), jnp.float32),
          pltpu.VMEM_SHARED((_NPAD, _MSG), jnp.float32),
      ],
  )
  def k(msg_hbm, dst_hbm, z_hbm, out0_hbm, out1_hbm,
        di_v, rows_v, acc_sh):
    cid = lax.axis_index("c")
    sid = lax.axis_index("s")
    wid = sid * _NC + cid
    row0 = wid * rpt

    pltpu.sync_copy(z_hbm, acc_sh.at[pl.ds(sid * _RPT, _RPT)])
    plsc.subcore_barrier()

    @pl.loop(0, rpt, step=_IDX_BLK)
    def _(j2):
      pltpu.sync_copy(dst_hbm.at[pl.ds(row0 + j2, _IDX_BLK)], di_v)

      @pl.loop(0, _IDX_BLK)
      def _(j):
        eoff = (row0 + j2 + j) * _S
        pltpu.sync_copy(msg_hbm.at[pl.ds(eoff, _S)], rows_v)
        pltpu.sync_copy(rows_v, acc_sh.at[di_v.at[j]], add=True)

    plsc.subcore_barrier()

    @pl.when(cid == 0)
    def _():
      pltpu.sync_copy(acc_sh.at[pl.ds(sid * _RPT, _RPT)],
                      out0_hbm.at[pl.ds(sid * _RPT, _RPT)])

    @pl.when(cid == 1)
    def _():
      pltpu.sync_copy(acc_sh.at[pl.ds(sid * _RPT, _RPT)],
                      out1_hbm.at[pl.ds(sid * _RPT, _RPT)])

  return k(msg, dst3, zrows)


def _edge_mlp(xi, xj, ea, W1a, W1b, W1c, b1, W2, b2, W3, b3):
  epad = xi.shape[0]
  grid = (epad // _BE,)

  def body(xi_ref, xj_ref, ea_ref, W1a_ref, W1b_ref, W1c_ref, b1_ref,
           W2_ref, b2_ref, W3_ref, b3_ref, out_ref):
    cat = jnp.concatenate([xi_ref[...], xj_ref[...], ea_ref[...]], axis=1)
    W1 = jnp.concatenate([W1a_ref[...], W1b_ref[...], W1c_ref[...]], axis=0)
    h = _bdot(cat, W1) + b1_ref[...]
    h = jnp.maximum(h, 0.0)
    h = jnp.maximum(_bdot(h, W2_ref[...]) + b2_ref[...], 0.0)
    out_ref[...] = _bdot(h, W3_ref[...]) + b3_ref[...]

  full = lambda shape: pl.BlockSpec(shape, lambda i: (0, 0))
  return pl.pallas_call(
      body,
      grid=grid,
      in_specs=[
          pl.BlockSpec((_BE, _NF), lambda i: (i, 0)),
          pl.BlockSpec((_BE, _NF), lambda i: (i, 0)),
          pl.BlockSpec((_BE, _EF), lambda i: (i, 0)),
          full((_NF, _HID)),
          full((_NF, _HID)),
          full((_EF, _HID)),
          full((1, _HID)),
          full((_HID, _HID)),
          full((1, _HID)),
          full((_HID, _MSG)),
          full((1, _MSG)),
      ],
      out_specs=pl.BlockSpec((_BE, _MSG), lambda i: (i, 0)),
      out_shape=jax.ShapeDtypeStruct((epad, _MSG), jnp.float32),
  )(xi, xj, ea, W1a, W1b, W1c, b1, W2, b2, W3, b3)


def _node_pool(x, a0, a1, a2, a3, batch3,
               Wn1a, Wn1b, bn1, Wn2, bn2, Wn3, bn3,
               Wg1, bg1, Wg2, bg2, Wg3, bg3):
  grid = (_N // _BN,)
  nsteps = _N // _BN

  def body(x_ref, a0_ref, a1_ref, a2_ref, a3_ref, b_ref,
           Wn1a_ref, Wn1b_ref, bn1_ref, Wn2_ref, bn2_ref, Wn3_ref, bn3_ref,
           Wg1_ref, bg1_ref, Wg2_ref, bg2_ref, Wg3_ref, bg3_ref,
           out_ref, sums_ref, cnt_ref):
    i = pl.program_id(0)

    @pl.when(i == 0)
    def _():
      sums_ref[...] = jnp.zeros_like(sums_ref)
      cnt_ref[...] = jnp.zeros_like(cnt_ref)

    aggr = (a0_ref[...] + a1_ref[...]) + (a2_ref[...] + a3_ref[...])
    cat = jnp.concatenate([x_ref[...], aggr], axis=1)
    Wn1 = jnp.concatenate([Wn1a_ref[...], Wn1b_ref[...]], axis=0)
    h = _bdot(cat, Wn1) + bn1_ref[...]
    h = jnp.maximum(h, 0.0)
    h = jnp.maximum(_bdot(h, Wn2_ref[...]) + bn2_ref[...], 0.0)
    h = _bdot(h, Wn3_ref[...]) + bn3_ref[...]

    bids = b_ref[0, 0, :]
    gids = lax.broadcasted_iota(jnp.int32, (1, _G), 1)
    oh = (bids[:, None] == gids).astype(jnp.float32)
    dn = (((0,), (0,)), ((), ()))
    h1 = h.astype(jnp.bfloat16).astype(jnp.float32)
    r1 = h - h1
    h2 = r1.astype(jnp.bfloat16).astype(jnp.float32)
    h3 = r1 - h2
    sums_ref[...] += (lax.dot_general(oh, h1, dn)
                      + lax.dot_general(oh, h2, dn)
                      + lax.dot_general(oh, h3, dn))
    ones = jnp.ones((_BN, _MSG), jnp.float32)
    cnt_ref[...] += lax.dot_general(oh, ones, dn)

    @pl.when(i == nsteps - 1)
    def _():
      pooled = sums_ref[...] / jnp.maximum(cnt_ref[...], 1.0)
      g = jnp.maximum(_bdot(pooled, Wg1_ref[...]) + bg1_ref[...], 0.0)
      g = jnp.maximum(_bdot(g, Wg2_ref[...]) + bg2_ref[...], 0.0)
      out_ref[...] = _bdot(g, Wg3_ref[...]) + bg3_ref[...]

  full = lambda shape: pl.BlockSpec(shape, lambda i: tuple(0 for _ in shape))
  return pl.pallas_call(
      body,
      grid=grid,
      in_specs=[
          pl.BlockSpec((_BN, _NF), lambda i: (i, 0)),
          pl.BlockSpec((_BN, _MSG), lambda i: (i, 0)),
          pl.BlockSpec((_BN, _MSG), lambda i: (i, 0)),
          pl.BlockSpec((_BN, _MSG), lambda i: (i, 0)),
          pl.BlockSpec((_BN, _MSG), lambda i: (i, 0)),
          pl.BlockSpec((1, 1, _BN), lambda i: (i, 0, 0)),
          full((_NF, _HID)),
          full((_MSG, _HID)),
          full((1, _HID)),
          full((_HID, _HID)),
          full((1, _HID)),
          full((_HID, _NH)),
          full((1, _NH)),
          full((_NH, _NH)),
          full((1, _NH)),
          full((_NH, _NH)),
          full((1, _NH)),
          full((_NH, _NP)),
          full((1, _NP)),
      ],
      out_specs=pl.BlockSpec((_G, _NP), lambda i: (0, 0)),
      out_shape=jax.ShapeDtypeStruct((_G, _NP), jnp.float32),
      scratch_shapes=[
          pltpu.VMEM((_G, _MSG), jnp.float32),
          pltpu.VMEM((_G, _MSG), jnp.float32),
      ],
  )(x, a0, a1, a2, a3, batch3,
    Wn1a, Wn1b, bn1, Wn2, bn2, Wn3, bn3,
    Wg1, bg1, Wg2, bg2, Wg3, bg3)


def kernel(x, edge_index, edge_attr, batch,
           msg_W1, msg_b1, msg_W2, msg_b2, msg_W3, msg_b3,
           node_W1, node_b1, node_W2, node_b2, node_W3, node_b3,
           glob_W1, glob_b1, glob_W2, glob_b2, glob_W3, glob_b3):
  src = edge_index[0]
  dst = edge_index[1]
  pad = _EPAD - _E
  src3 = jnp.concatenate(
      [src, jnp.zeros((pad,), jnp.int32)]).reshape(_EPAD // _S, _S)
  dst3g = jnp.concatenate(
      [dst, jnp.zeros((pad,), jnp.int32)]).reshape(_EPAD // _S, _S)
  dst3s = jnp.concatenate(
      [dst, jnp.full((pad,), _DEAD_ROW, jnp.int32)]).reshape(_EPAD // _S, _S)
  ea = jnp.concatenate([edge_attr, jnp.zeros((pad, _EF), jnp.float32)], axis=0)

  hr = (_EPAD // _S) // 2
  he = _EPAD // 2
  margs = (msg_W1[:_NF], msg_W1[_NF:2 * _NF], msg_W1[2 * _NF:],
           msg_b1.reshape(1, _HID),
           msg_W2, msg_b2.reshape(1, _HID),
           msg_W3, msg_b3.reshape(1, _MSG))
  zrows = jnp.zeros((_RPT, _MSG), jnp.float32)

  xi0, xj0 = _sc_gather(x, src3[:hr], dst3g[:hr])
  xi1, xj1 = _sc_gather(x, src3[hr:], dst3g[hr:])
  msg0 = _edge_mlp(xi0, xj0, ea[:he], *margs)
  msg1 = _edge_mlp(xi1, xj1, ea[he:], *margs)
  a0, a1 = _sc_scatter_add(msg0, dst3s[:hr], zrows)
  a2, a3 = _sc_scatter_add(msg1, dst3s[hr:], zrows)

  out = _node_pool(
      x, a0, a1, a2, a3, batch.reshape(_N // _BN, 1, _BN),
      node_W1[:_NF], node_W1[_NF:], node_b1.reshape(1, _HID),
      node_W2, node_b2.reshape(1, _HID),
      node_W3, node_b3.reshape(1, _NH),
      glob_W1, glob_b1.reshape(1, _NH),
      glob_W2, glob_b2.reshape(1, _NH),
      glob_W3, glob_b3.reshape(1, _NP))
  return out
